# baseline (speedup 1.0000x reference)
import functools

import jax
import jax.numpy as jnp
from jax import lax
from jax.experimental import pallas as pl
from jax.experimental.pallas import tpu as pltpu
from jax.experimental.pallas import tpu_sc as plsc

N = 10000
E = 320000
H = 200
HH = 128
HP = 2 * HH
R2 = 400

NC, NS = 2, 16
CH = 120
NCHUNK = 168
NBUF = 3
NGROUP = NCHUNK // NBUF
EP = NS * NCHUNK * CH
NA = N + 8
RSTRIDE = 624
RSPAN = 640

_NEG_SLOPE = (1.0 / 8.0 + 1.0 / 3.0) / 2.0


def _segsum_body(grow, table, sd, zeros, out, sd0, sd1, sd2, r0, r1, r2,
                 i0, i1, i2, g0, g1, g2, s0, s1, s2, acc):
    sdb = (sd0, sd1, sd2)
    rows = (r0, r1, r2)
    isem = (i0, i1, i2)
    gsem = (g0, g1, g2)
    ssem = (s0, s1, s2)
    c = lax.axis_index("c")
    s = lax.axis_index("s")

    pltpu.sync_copy(zeros, acc.at[pl.ds(s * RSTRIDE, RSPAN)])
    plsc.subcore_barrier()

    for b in range(NBUF):
        pltpu.async_copy(sd.at[c, s, b], sdb[b], isem[b])

    def _window(w, _):
        j0 = w * NBUF
        for b in range(NBUF):
            pltpu.make_async_copy(sd.at[c, s, j0 + b], sdb[b],
                                  isem[b]).wait()
            pltpu.async_copy(table.at[sdb[b].at[grow]], rows[b], gsem[b])
        for b in range(NBUF):
            pltpu.make_async_copy(table.at[sdb[b].at[grow]], rows[b],
                                  gsem[b]).wait()
            pltpu.async_copy(rows[b], acc.at[sdb[b].at[2]], ssem[b],
                             add=True)
        @pl.when(w < NGROUP - 1)
        def _():
            for b in range(NBUF):
                pltpu.make_async_copy(rows[b], acc.at[sdb[b].at[2]],
                                      ssem[b]).wait()
                pltpu.async_copy(sd.at[c, s, j0 + NBUF + b], sdb[b],
                                 isem[b])
        return 0

    lax.fori_loop(0, NGROUP, _window, 0)
    for b in range(NBUF):
        pltpu.make_async_copy(rows[b], acc.at[sdb[b].at[2]], ssem[b]).wait()
    plsc.subcore_barrier()

    pltpu.sync_copy(acc.at[pl.ds(s * RSTRIDE, RSPAN)],
                    out.at[pl.ds(c * N + s * RSTRIDE, RSPAN)])


def _segsum(table2, sd, zeros, grow):
    table2, sd, zeros = lax.optimization_barrier((table2, sd, zeros))
    return pl.kernel(
        functools.partial(_segsum_body, grow),
        out_type=jax.ShapeDtypeStruct((2 * N, HH), jnp.float32),
        mesh=plsc.VectorSubcoreMesh(core_axis_name="c", subcore_axis_name="s"),
        scratch_types=(
            [pltpu.VMEM((3, CH), jnp.int32)] * NBUF
            + [pltpu.VMEM((CH, HH), jnp.float32)] * NBUF
            + [pltpu.SemaphoreType.DMA] * (3 * NBUF)
            + [pltpu.VMEM_SHARED((NA, HH), jnp.float32)]
        ),
        name="edge_segsum",
    )(table2, sd, zeros)


_BN = 2000


def _l2norm(x):
    n = jnp.sqrt(jnp.sum(x * x, axis=-1, keepdims=True))
    return x / jnp.clip(n, 1e-12, None)


def _norm_body(x_ref, o0_ref, o1_ref):
    y = _l2norm(x_ref[...])
    o0_ref[...] = y[:, :HH]
    o1_ref[...] = jnp.concatenate(
        [y[:, HH:], jnp.zeros((y.shape[0], HP - H), jnp.float32)], axis=1)


def _deg_inv(r1):
    col = lax.broadcasted_iota(jnp.int32, r1.shape, 1)
    deg = jnp.sum(jnp.where(col >= H - HH, r1, 0.0), axis=1) * (1.0 / (HP - H))
    return 1.0 / jnp.maximum(deg, 1.0)


def _merge_q(p0, p1, r0, r1):
    q = jnp.concatenate([p0[...] + r0[...], p1[...] + r1[...]], axis=1)
    return q, _deg_inv(r1[...])


def _layer_body(p0, p1, r0, r1, h0, h1, wn_ref, wl_ref, o0_ref, o1_ref):
    q, inv = _merge_q(p0, p1, r0, r1)
    agg = jnp.dot(q, wn_ref[...], preferred_element_type=jnp.float32)
    agg = agg * inv[:, None]
    h = jnp.concatenate([h0[...], h1[...]], axis=1)
    x = agg + jnp.dot(h, wl_ref[...], preferred_element_type=jnp.float32)
    y = jnp.where(x >= 0, x, x * _NEG_SLOPE)
    o0_ref[...] = y[:, :HH]
    o1_ref[...] = y[:, HH:]


def _final_body(p0, p1, r0, r1, h10, h11, h00, h01,
                wn_ref, wl_ref, wt_ref, b_ref, o_ref):
    q, inv = _merge_q(p0, p1, r0, r1)
    agg = jnp.dot(q, wn_ref[...], preferred_element_type=jnp.float32)
    agg = agg * inv[:, None]
    h1 = jnp.concatenate([h10[...], h11[...]], axis=1)
    x = agg + jnp.dot(h1, wl_ref[...], preferred_element_type=jnp.float32)
    h2 = jnp.where(x >= 0, x, x * _NEG_SLOPE)
    h2 = _l2norm(h2)
    h0 = jnp.concatenate([h00[...], h01[...]], axis=1)
    g = jax.nn.sigmoid(
        jnp.dot(h0, wt_ref[...], preferred_element_type=jnp.float32)
        + b_ref[...])
    o_ref[...] = g * h2 + (1.0 - g) * h0


def _half_spec():
    return pl.BlockSpec((_BN, HH), lambda i: (i, 0))


def _full_spec(shape):
    return pl.BlockSpec(shape, lambda i: tuple(0 for _ in shape))


def _norm_call(x):
    return pl.pallas_call(
        _norm_body,
        grid=(N // _BN,),
        in_specs=[pl.BlockSpec((_BN, H), lambda i: (i, 0))],
        out_specs=[_half_spec(), _half_spec()],
        out_shape=[jax.ShapeDtypeStruct((N, HH), jnp.float32),
                   jax.ShapeDtypeStruct((N, HH), jnp.float32)],
    )(x)


def _layer_call(p, r, h, wn, wl):
    return pl.pallas_call(
        _layer_body,
        grid=(N // _BN,),
        in_specs=[_half_spec()] * 6 + [_full_spec((HP, HP))] * 2,
        out_specs=[_half_spec(), _half_spec()],
        out_shape=[jax.ShapeDtypeStruct((N, HH), jnp.float32),
                   jax.ShapeDtypeStruct((N, HH), jnp.float32)],
    )(p[:N], p[N:], r[:N], r[N:], h[0], h[1], wn, wl)


def _final_call(p, r, h1, h0, wn, wl, wt, b):
    return pl.pallas_call(
        _final_body,
        grid=(N // _BN,),
        in_specs=([_half_spec()] * 8
                  + [_full_spec((HP, HP))] * 3 + [_full_spec((1, HP))]),
        out_specs=pl.BlockSpec((_BN, HP), lambda i: (i, 0)),
        out_shape=jax.ShapeDtypeStruct((N, HP), jnp.float32),
    )(p[:N], p[N:], r[:N], r[N:], h1[0], h1[1], h0[0], h0[1],
      wn, wl, wt, b)


def kernel(dynamic_emb, emb_rel, W_neigh, W_loop, time_gate_weight,
           time_gate_bias, edge_index, edge_type):
    f32 = jnp.float32
    src = edge_index[0].astype(jnp.int32)
    dst = edge_index[1].astype(jnp.int32)
    etype = edge_type.astype(jnp.int32)

    pad_w = lambda w: jnp.pad(w.astype(f32), ((0, HP - H), (0, HP - H)))
    wn0, wn1 = pad_w(W_neigh[0]), pad_w(W_neigh[1])
    wl0, wl1 = pad_w(W_loop[0]), pad_w(W_loop[1])
    wt = pad_w(time_gate_weight)
    bias = jnp.pad(time_gate_bias.astype(f32), (0, HP - H))[None, :]

    h0 = _norm_call(dynamic_emb.astype(f32))
    rel = emb_rel.astype(f32)
    rel2 = jnp.concatenate(
        [rel[:, :HH],
         jnp.concatenate([rel[:, HH:], jnp.ones((R2, HP - H), f32)], axis=1)],
        axis=0)

    srcp = jnp.pad(src, (0, EP - E)).reshape(NS, NCHUNK, CH)
    etyp = jnp.pad(etype, (0, EP - E)).reshape(NS, NCHUNK, CH)
    dstp = jnp.pad(dst, (0, EP - E),
                   constant_values=N).reshape(NS, NCHUNK, CH)
    sd = jnp.stack(
        [jnp.stack([srcp, etyp, dstp], axis=2),
         jnp.stack([srcp + N, etyp + R2, dstp], axis=2)])
    zeros = jnp.zeros((RSPAN, HH), f32)

    r = _segsum(rel2, sd, zeros, 1)
    p0 = _segsum(jnp.concatenate([h0[0], h0[1]], axis=0), sd, zeros, 0)
    h1 = _layer_call(p0, r, h0, wn0, wl0)
    p1 = _segsum(jnp.concatenate([h1[0], h1[1]], axis=0), sd, zeros, 0)
    out = _final_call(p1, r, h1, h0, wn1, wl1, wt, bias)
    return out[:, :H]


# speedup vs baseline: 1.0215x; 1.0215x over previous
import functools

import jax
import jax.numpy as jnp
from jax import lax
from jax.experimental import pallas as pl
from jax.experimental.pallas import tpu as pltpu
from jax.experimental.pallas import tpu_sc as plsc

N = 10000
E = 320000
H = 200
HH = 128
HP = 2 * HH
R2 = 400

NC, NS = 2, 16
CH = 120
NCHUNK = 168
NBUF = 3
NGROUP = NCHUNK // NBUF
EP = NS * NCHUNK * CH
NA = N + 8
RSTRIDE = 624
RSPAN = 640

_NEG_SLOPE = (1.0 / 8.0 + 1.0 / 3.0) / 2.0


def _segsum_body(table, sd, zeros, out, sd0, sd1, sd2, r0, r1, r2,
                 i0, i1, i2, g0, g1, g2, s0, s1, s2, acc):
    sdb = (sd0, sd1, sd2)
    rows = (r0, r1, r2)
    isem = (i0, i1, i2)
    gsem = (g0, g1, g2)
    ssem = (s0, s1, s2)
    c = lax.axis_index("c")
    s = lax.axis_index("s")

    pltpu.sync_copy(zeros, acc.at[pl.ds(s * RSTRIDE, RSPAN)])
    plsc.subcore_barrier()

    for b in range(NBUF):
        pltpu.async_copy(sd.at[c, s, b], sdb[b], isem[b])

    def _window(w, _):
        j0 = w * NBUF
        for b in range(NBUF):
            pltpu.make_async_copy(sd.at[c, s, j0 + b], sdb[b],
                                  isem[b]).wait()
            pltpu.async_copy(table.at[sdb[b].at[0]], rows[b], gsem[b])
        for b in range(NBUF):
            pltpu.make_async_copy(table.at[sdb[b].at[0]], rows[b],
                                  gsem[b]).wait()
            pltpu.async_copy(rows[b], acc.at[sdb[b].at[1]], ssem[b],
                             add=True)
        @pl.when(w < NGROUP - 1)
        def _():
            for b in range(NBUF):
                pltpu.make_async_copy(rows[b], acc.at[sdb[b].at[1]],
                                      ssem[b]).wait()
                pltpu.async_copy(sd.at[c, s, j0 + NBUF + b], sdb[b],
                                 isem[b])
        return 0

    lax.fori_loop(0, NGROUP, _window, 0)
    for b in range(NBUF):
        pltpu.make_async_copy(rows[b], acc.at[sdb[b].at[1]], ssem[b]).wait()
    plsc.subcore_barrier()

    pltpu.sync_copy(acc.at[pl.ds(s * RSTRIDE, RSPAN)],
                    out.at[pl.ds(c * N + s * RSTRIDE, RSPAN)])


def _segsum(table2, sd, zeros):
    table2, sd, zeros = lax.optimization_barrier((table2, sd, zeros))
    return pl.kernel(
        _segsum_body,
        out_type=jax.ShapeDtypeStruct((2 * N, HH), jnp.float32),
        mesh=plsc.VectorSubcoreMesh(core_axis_name="c", subcore_axis_name="s"),
        scratch_types=(
            [pltpu.VMEM((2, CH), jnp.int32)] * NBUF
            + [pltpu.VMEM((CH, HH), jnp.float32)] * NBUF
            + [pltpu.SemaphoreType.DMA] * (3 * NBUF)
            + [pltpu.VMEM_SHARED((NA, HH), jnp.float32)]
        ),
        name="edge_segsum",
    )(table2, sd, zeros)


_BN = 2000


def _l2norm(x):
    n = jnp.sqrt(jnp.sum(x * x, axis=-1, keepdims=True))
    return x / jnp.clip(n, 1e-12, None)


def _norm_body(x_ref, o0_ref, o1_ref):
    y = _l2norm(x_ref[...])
    o0_ref[...] = y[:, :HH]
    o1_ref[...] = jnp.concatenate(
        [y[:, HH:], jnp.zeros((y.shape[0], HP - H), jnp.float32)], axis=1)


def _deg_inv(r1):
    col = lax.broadcasted_iota(jnp.int32, r1.shape, 1)
    deg = jnp.sum(jnp.where(col >= H - HH, r1, 0.0), axis=1) * (1.0 / (HP - H))
    return 1.0 / jnp.maximum(deg, 1.0)


def _merge_q(p0, p1, r0, r1):
    q = jnp.concatenate([p0[...] + r0[...], p1[...] + r1[...]], axis=1)
    return q, _deg_inv(r1[...])


def _layer_body(p0, p1, r0, r1, h0, h1, wn_ref, wl_ref, o0_ref, o1_ref):
    q, inv = _merge_q(p0, p1, r0, r1)
    agg = jnp.dot(q, wn_ref[...], preferred_element_type=jnp.float32)
    agg = agg * inv[:, None]
    h = jnp.concatenate([h0[...], h1[...]], axis=1)
    x = agg + jnp.dot(h, wl_ref[...], preferred_element_type=jnp.float32)
    y = jnp.where(x >= 0, x, x * _NEG_SLOPE)
    o0_ref[...] = y[:, :HH]
    o1_ref[...] = y[:, HH:]


def _final_body(p0, p1, r0, r1, h10, h11, h00, h01,
                wn_ref, wl_ref, wt_ref, b_ref, o_ref):
    q, inv = _merge_q(p0, p1, r0, r1)
    agg = jnp.dot(q, wn_ref[...], preferred_element_type=jnp.float32)
    agg = agg * inv[:, None]
    h1 = jnp.concatenate([h10[...], h11[...]], axis=1)
    x = agg + jnp.dot(h1, wl_ref[...], preferred_element_type=jnp.float32)
    h2 = jnp.where(x >= 0, x, x * _NEG_SLOPE)
    h2 = _l2norm(h2)
    h0 = jnp.concatenate([h00[...], h01[...]], axis=1)
    g = jax.nn.sigmoid(
        jnp.dot(h0, wt_ref[...], preferred_element_type=jnp.float32)
        + b_ref[...])
    o_ref[...] = g * h2 + (1.0 - g) * h0


def _half_spec():
    return pl.BlockSpec((_BN, HH), lambda i: (i, 0))


def _full_spec(shape):
    return pl.BlockSpec(shape, lambda i: tuple(0 for _ in shape))


def _norm_call(x):
    return pl.pallas_call(
        _norm_body,
        grid=(N // _BN,),
        in_specs=[pl.BlockSpec((_BN, H), lambda i: (i, 0))],
        out_specs=[_half_spec(), _half_spec()],
        out_shape=[jax.ShapeDtypeStruct((N, HH), jnp.float32),
                   jax.ShapeDtypeStruct((N, HH), jnp.float32)],
    )(x)


def _layer_call(p, r, h, wn, wl):
    return pl.pallas_call(
        _layer_body,
        grid=(N // _BN,),
        in_specs=[_half_spec()] * 6 + [_full_spec((HP, HP))] * 2,
        out_specs=[_half_spec(), _half_spec()],
        out_shape=[jax.ShapeDtypeStruct((N, HH), jnp.float32),
                   jax.ShapeDtypeStruct((N, HH), jnp.float32)],
    )(p[:N], p[N:], r[:N], r[N:], h[0], h[1], wn, wl)


def _final_call(p, r, h1, h0, wn, wl, wt, b):
    return pl.pallas_call(
        _final_body,
        grid=(N // _BN,),
        in_specs=([_half_spec()] * 8
                  + [_full_spec((HP, HP))] * 3 + [_full_spec((1, HP))]),
        out_specs=pl.BlockSpec((_BN, HP), lambda i: (i, 0)),
        out_shape=jax.ShapeDtypeStruct((N, HP), jnp.float32),
    )(p[:N], p[N:], r[:N], r[N:], h1[0], h1[1], h0[0], h0[1],
      wn, wl, wt, b)


def kernel(dynamic_emb, emb_rel, W_neigh, W_loop, time_gate_weight,
           time_gate_bias, edge_index, edge_type):
    f32 = jnp.float32
    src = edge_index[0].astype(jnp.int32)
    dst = edge_index[1].astype(jnp.int32)
    etype = edge_type.astype(jnp.int32)

    pad_w = lambda w: jnp.pad(w.astype(f32), ((0, HP - H), (0, HP - H)))
    wn0, wn1 = pad_w(W_neigh[0]), pad_w(W_neigh[1])
    wl0, wl1 = pad_w(W_loop[0]), pad_w(W_loop[1])
    wt = pad_w(time_gate_weight)
    bias = jnp.pad(time_gate_bias.astype(f32), (0, HP - H))[None, :]

    h0 = _norm_call(dynamic_emb.astype(f32))
    rel = emb_rel.astype(f32)
    rel2 = jnp.concatenate(
        [rel[:, :HH],
         jnp.concatenate([rel[:, HH:], jnp.ones((R2, HP - H), f32)], axis=1)],
        axis=0)

    srcp = jnp.pad(src, (0, EP - E)).reshape(NS, NCHUNK, CH)
    etyp = jnp.pad(etype, (0, EP - E)).reshape(NS, NCHUNK, CH)
    dstp = jnp.pad(dst, (0, EP - E),
                   constant_values=N).reshape(NS, NCHUNK, CH)
    sd_of = lambda g, voff: jnp.stack(
        [jnp.stack([g, dstp], axis=2), jnp.stack([g + voff, dstp], axis=2)])
    sd_h = sd_of(srcp, N)
    sd_r = sd_of(etyp, R2)
    zeros = jnp.zeros((RSPAN, HH), f32)

    r = _segsum(rel2, sd_r, zeros)
    p0 = _segsum(jnp.concatenate([h0[0], h0[1]], axis=0), sd_h, zeros)
    h1 = _layer_call(p0, r, h0, wn0, wl0)
    p1 = _segsum(jnp.concatenate([h1[0], h1[1]], axis=0), sd_h, zeros)
    out = _final_call(p1, r, h1, h0, wn1, wl1, wt, bias)
    return out[:, :H]


# speedup vs baseline: 1.2212x; 1.1955x over previous
import functools

import jax
import jax.numpy as jnp
from jax import lax
from jax.experimental import pallas as pl
from jax.experimental.pallas import tpu as pltpu
from jax.experimental.pallas import tpu_sc as plsc

N = 10000
E = 320000
H = 200
HH = 128
HP = 2 * HH
R2 = 400

NC, NS = 2, 16
CH = 88
NCHUNK = 228
NBUF = 4
NGROUP = NCHUNK // NBUF
EP = NS * NCHUNK * CH
NA = N + 8
RSTRIDE = 624
RSPAN = 640

_NEG_SLOPE = (1.0 / 8.0 + 1.0 / 3.0) / 2.0


def _segsum_body(table, sd, zeros, out, sd0, sd1, sd2, sd3, r0, r1, r2, r3,
                 i0, i1, i2, i3, g0, g1, g2, g3, s0, s1, s2, s3, acc):
    sdb = (sd0, sd1, sd2, sd3)
    rows = (r0, r1, r2, r3)
    isem = (i0, i1, i2, i3)
    gsem = (g0, g1, g2, g3)
    ssem = (s0, s1, s2, s3)
    c = lax.axis_index("c")
    s = lax.axis_index("s")

    pltpu.sync_copy(zeros, acc.at[pl.ds(s * RSTRIDE, RSPAN)])
    plsc.subcore_barrier()

    for b in range(NBUF):
        pltpu.async_copy(sd.at[c, s, b], sdb[b], isem[b])

    def _window(w, _):
        j0 = w * NBUF
        for b in range(NBUF):
            pltpu.make_async_copy(sd.at[c, s, j0 + b], sdb[b],
                                  isem[b]).wait()
            pltpu.async_copy(table.at[sdb[b].at[0]], rows[b], gsem[b])
        for b in range(NBUF):
            pltpu.make_async_copy(table.at[sdb[b].at[0]], rows[b],
                                  gsem[b]).wait()
            pltpu.async_copy(rows[b], acc.at[sdb[b].at[1]], ssem[b],
                             add=True)
        @pl.when(w < NGROUP - 1)
        def _():
            for b in range(NBUF):
                pltpu.make_async_copy(rows[b], acc.at[sdb[b].at[1]],
                                      ssem[b]).wait()
                pltpu.async_copy(sd.at[c, s, j0 + NBUF + b], sdb[b],
                                 isem[b])
        return 0

    lax.fori_loop(0, NGROUP, _window, 0)
    for b in range(NBUF):
        pltpu.make_async_copy(rows[b], acc.at[sdb[b].at[1]], ssem[b]).wait()
    plsc.subcore_barrier()

    pltpu.sync_copy(acc.at[pl.ds(s * RSTRIDE, RSPAN)],
                    out.at[pl.ds(c * N + s * RSTRIDE, RSPAN)])


def _segsum(table2, sd, zeros):
    table2, sd, zeros = lax.optimization_barrier((table2, sd, zeros))
    return pl.kernel(
        _segsum_body,
        out_type=jax.ShapeDtypeStruct((2 * N, HH), jnp.float32),
        mesh=plsc.VectorSubcoreMesh(core_axis_name="c", subcore_axis_name="s"),
        scratch_types=(
            [pltpu.VMEM((2, CH), jnp.int32)] * NBUF
            + [pltpu.VMEM((CH, HH), jnp.float32)] * NBUF
            + [pltpu.SemaphoreType.DMA] * (3 * NBUF)
            + [pltpu.VMEM_SHARED((NA, HH), jnp.float32)]
        ),
        name="edge_segsum",
    )(table2, sd, zeros)


_BN = 2000


def _l2norm(x):
    n = jnp.sqrt(jnp.sum(x * x, axis=-1, keepdims=True))
    return x / jnp.clip(n, 1e-12, None)


def _norm_body(x_ref, o0_ref, o1_ref):
    y = _l2norm(x_ref[...])
    o0_ref[...] = y[:, :HH]
    o1_ref[...] = jnp.concatenate(
        [y[:, HH:], jnp.zeros((y.shape[0], HP - H), jnp.float32)], axis=1)


def _deg_inv(r1):
    col = lax.broadcasted_iota(jnp.int32, r1.shape, 1)
    deg = jnp.sum(jnp.where(col >= H - HH, r1, 0.0), axis=1) * (1.0 / (HP - H))
    return 1.0 / jnp.maximum(deg, 1.0)


def _merge_q(p0, p1, r0, r1):
    q = jnp.concatenate([p0[...] + r0[...], p1[...] + r1[...]], axis=1)
    return q, _deg_inv(r1[...])


def _layer_body(p0, p1, r0, r1, h0, h1, wn_ref, wl_ref, o0_ref, o1_ref):
    q, inv = _merge_q(p0, p1, r0, r1)
    agg = jnp.dot(q, wn_ref[...], preferred_element_type=jnp.float32)
    agg = agg * inv[:, None]
    h = jnp.concatenate([h0[...], h1[...]], axis=1)
    x = agg + jnp.dot(h, wl_ref[...], preferred_element_type=jnp.float32)
    y = jnp.where(x >= 0, x, x * _NEG_SLOPE)
    o0_ref[...] = y[:, :HH]
    o1_ref[...] = y[:, HH:]


def _final_body(p0, p1, r0, r1, h10, h11, h00, h01,
                wn_ref, wl_ref, wt_ref, b_ref, o_ref):
    q, inv = _merge_q(p0, p1, r0, r1)
    agg = jnp.dot(q, wn_ref[...], preferred_element_type=jnp.float32)
    agg = agg * inv[:, None]
    h1 = jnp.concatenate([h10[...], h11[...]], axis=1)
    x = agg + jnp.dot(h1, wl_ref[...], preferred_element_type=jnp.float32)
    h2 = jnp.where(x >= 0, x, x * _NEG_SLOPE)
    h2 = _l2norm(h2)
    h0 = jnp.concatenate([h00[...], h01[...]], axis=1)
    g = jax.nn.sigmoid(
        jnp.dot(h0, wt_ref[...], preferred_element_type=jnp.float32)
        + b_ref[...])
    o_ref[...] = g * h2 + (1.0 - g) * h0


def _half_spec():
    return pl.BlockSpec((_BN, HH), lambda i: (i, 0))


def _full_spec(shape):
    return pl.BlockSpec(shape, lambda i: tuple(0 for _ in shape))


def _norm_call(x):
    return pl.pallas_call(
        _norm_body,
        grid=(N // _BN,),
        in_specs=[pl.BlockSpec((_BN, H), lambda i: (i, 0))],
        out_specs=[_half_spec(), _half_spec()],
        out_shape=[jax.ShapeDtypeStruct((N, HH), jnp.float32),
                   jax.ShapeDtypeStruct((N, HH), jnp.float32)],
    )(x)


def _layer_call(p, r, h, wn, wl):
    return pl.pallas_call(
        _layer_body,
        grid=(N // _BN,),
        in_specs=[_half_spec()] * 6 + [_full_spec((HP, HP))] * 2,
        out_specs=[_half_spec(), _half_spec()],
        out_shape=[jax.ShapeDtypeStruct((N, HH), jnp.float32),
                   jax.ShapeDtypeStruct((N, HH), jnp.float32)],
    )(p[:N], p[N:], r[:N], r[N:], h[0], h[1], wn, wl)


def _final_call(p, r, h1, h0, wn, wl, wt, b):
    return pl.pallas_call(
        _final_body,
        grid=(N // _BN,),
        in_specs=([_half_spec()] * 8
                  + [_full_spec((HP, HP))] * 3 + [_full_spec((1, HP))]),
        out_specs=pl.BlockSpec((_BN, HP), lambda i: (i, 0)),
        out_shape=jax.ShapeDtypeStruct((N, HP), jnp.float32),
    )(p[:N], p[N:], r[:N], r[N:], h1[0], h1[1], h0[0], h0[1],
      wn, wl, wt, b)


def kernel(dynamic_emb, emb_rel, W_neigh, W_loop, time_gate_weight,
           time_gate_bias, edge_index, edge_type):
    f32 = jnp.float32
    src = edge_index[0].astype(jnp.int32)
    dst = edge_index[1].astype(jnp.int32)
    etype = edge_type.astype(jnp.int32)

    pad_w = lambda w: jnp.pad(w.astype(f32), ((0, HP - H), (0, HP - H)))
    wn0, wn1 = pad_w(W_neigh[0]), pad_w(W_neigh[1])
    wl0, wl1 = pad_w(W_loop[0]), pad_w(W_loop[1])
    wt = pad_w(time_gate_weight)
    bias = jnp.pad(time_gate_bias.astype(f32), (0, HP - H))[None, :]

    h0 = _norm_call(dynamic_emb.astype(f32))
    rel = emb_rel.astype(f32)
    rel2 = jnp.concatenate(
        [rel[:, :HH],
         jnp.concatenate([rel[:, HH:], jnp.ones((R2, HP - H), f32)], axis=1)],
        axis=0)

    srcp = jnp.pad(src, (0, EP - E)).reshape(NS, NCHUNK, CH)
    etyp = jnp.pad(etype, (0, EP - E)).reshape(NS, NCHUNK, CH)
    dstp = jnp.pad(dst, (0, EP - E),
                   constant_values=N).reshape(NS, NCHUNK, CH)
    sd_of = lambda g, voff: jnp.stack(
        [jnp.stack([g, dstp], axis=2), jnp.stack([g + voff, dstp], axis=2)])
    sd_h = sd_of(srcp, N)
    sd_r = sd_of(etyp, R2)
    zeros = jnp.zeros((RSPAN, HH), f32)

    r = _segsum(rel2, sd_r, zeros)
    p0 = _segsum(jnp.concatenate([h0[0], h0[1]], axis=0), sd_h, zeros)
    h1 = _layer_call(p0, r, h0, wn0, wl0)
    p1 = _segsum(jnp.concatenate([h1[0], h1[1]], axis=0), sd_h, zeros)
    out = _final_call(p1, r, h1, h0, wn1, wl1, wt, bias)
    return out[:, :H]
